# baseline (speedup 1.0000x reference)
import functools

import jax
import jax.numpy as jnp
from jax import lax
from jax.experimental import pallas as pl
from jax.experimental.pallas import tpu as pltpu
from jax.experimental.pallas import tpu_sc as plsc

_B = 16384
_C = 1000
_NC = 2
_NS = 16
_NW = _NC * _NS
_BPW = _B // _NW
_CHUNK = 128
_NCHUNK = _BPW // _CHUNK
_LANES = 16


@functools.partial(
    pl.kernel,
    out_type=jax.ShapeDtypeStruct((_B,), jnp.float32),
    mesh=plsc.VectorSubcoreMesh(
        core_axis_name="c", subcore_axis_name="s",
        num_cores=_NC, num_subcores=_NS),
    scratch_types=[
        pltpu.VMEM((_BPW,), jnp.int32),
        pltpu.VMEM((_BPW,), jnp.int32),
        pltpu.VMEM((_BPW,), jnp.float32),
        pltpu.SemaphoreType.DMA,
    ],
)
def _match_class_sc(table_hbm, idx_hbm, out_hbm, idx_v, flat_v, vals_v, sem):
    wid = lax.axis_index("s") * _NC + lax.axis_index("c")
    base = wid * _BPW
    pltpu.sync_copy(idx_hbm.at[pl.ds(base, _BPW)], idx_v)
    lane_off = lax.iota(jnp.int32, _LANES) * _C
    for i in range(_BPW // _LANES):
        row0 = (base + i * _LANES) * _C
        flat_v[pl.ds(i * _LANES, _LANES)] = (
            idx_v[pl.ds(i * _LANES, _LANES)] + lane_off + row0
        )
    copies = [
        pltpu.async_copy(
            table_hbm.at[flat_v.at[pl.ds(j * _CHUNK, _CHUNK)]],
            vals_v.at[pl.ds(j * _CHUNK, _CHUNK)],
            sem,
        )
        for j in range(_NCHUNK)
    ]
    for cp in copies:
        cp.wait()
    pltpu.sync_copy(vals_v, out_hbm.at[pl.ds(base, _BPW)])


def kernel(class_pred_softmax, class_max_prob_A_index):
    table = class_pred_softmax.reshape(-1)
    idx = class_max_prob_A_index.astype(jnp.int32)
    return _match_class_sc(table, idx)


# speedup vs baseline: 7.0142x; 7.0142x over previous
import functools

import jax
import jax.numpy as jnp
from jax import lax
from jax.experimental import pallas as pl
from jax.experimental.pallas import tpu as pltpu
from jax.experimental.pallas import tpu_sc as plsc

_B = 16384
_C = 1000
_NC = 2
_NS = 16
_NW = _NC * _NS
_BPW = _B // _NW
_CHUNK = 128
_NCHUNK = _BPW // _CHUNK
_LANES = 16


@functools.partial(
    pl.kernel,
    out_type=jax.ShapeDtypeStruct((_B,), jnp.float32),
    mesh=plsc.VectorSubcoreMesh(
        core_axis_name="c", subcore_axis_name="s",
        num_cores=_NC, num_subcores=_NS),
    scratch_types=[
        pltpu.VMEM((_BPW,), jnp.int32),
        pltpu.VMEM((_BPW,), jnp.int32),
        pltpu.VMEM((_BPW,), jnp.float32),
        pltpu.SemaphoreType.DMA,
    ],
)
def _match_class_sc(table_hbm, idx_hbm, out_hbm, idx_v, flat_v, vals_v, sem):
    wid = lax.axis_index("s") * _NC + lax.axis_index("c")
    base = wid * _BPW
    pltpu.sync_copy(idx_hbm.at[pl.ds(base, _BPW)], idx_v)
    lane = lax.iota(jnp.int32, _LANES)
    for i in range(_BPW // _LANES):
        r = base + i * _LANES + lane
        c = idx_v[pl.ds(i * _LANES, _LANES)]
        w = (
            lax.shift_left(lax.shift_right_logical(c, 3), 17)
            + lax.shift_left(lax.shift_right_logical(r, 7), 10)
            + lax.shift_left(lax.bitwise_and(c, 7), 7)
            + lax.bitwise_and(r, 127)
        )
        flat_v[pl.ds(i * _LANES, _LANES)] = w
    copies = [
        pltpu.async_copy(
            table_hbm.at[flat_v.at[pl.ds(j * _CHUNK, _CHUNK)]],
            vals_v.at[pl.ds(j * _CHUNK, _CHUNK)],
            sem,
        )
        for j in range(_NCHUNK)
    ]
    for cp in copies:
        cp.wait()
    pltpu.sync_copy(vals_v, out_hbm.at[pl.ds(base, _BPW)])


def kernel(class_pred_softmax, class_max_prob_A_index):
    x = class_pred_softmax.T.reshape(_C // 8, 8, _B // 128, 128)
    x = x.transpose(0, 2, 1, 3)
    flat = x.reshape(_B * _C)
    idx = class_max_prob_A_index.astype(jnp.int32)
    return _match_class_sc(flat, idx)


# speedup vs baseline: 7.0474x; 1.0047x over previous
import functools

import jax
import jax.numpy as jnp
from jax import lax
from jax.experimental import pallas as pl
from jax.experimental.pallas import tpu as pltpu
from jax.experimental.pallas import tpu_sc as plsc

_B = 16384
_C = 1000
_NC = 2
_NS = 16
_NW = _NC * _NS
_BPW = _B // _NW
_CHUNK = 128
_NCHUNK = _BPW // _CHUNK
_LANES = 16


@functools.partial(
    pl.kernel,
    out_type=jax.ShapeDtypeStruct((_B,), jnp.float32),
    mesh=plsc.VectorSubcoreMesh(
        core_axis_name="c", subcore_axis_name="s",
        num_cores=_NC, num_subcores=_NS),
    scratch_types=[
        pltpu.VMEM((_BPW,), jnp.int32),
        pltpu.VMEM((_BPW,), jnp.int32),
        pltpu.VMEM((_BPW,), jnp.float32),
        pltpu.SemaphoreType.DMA,
        pltpu.SemaphoreType.DMA,
    ],
)
def _match_class_sc(table_hbm, idx_hbm, out_hbm,
                    idx_v, flat_v, vals_v, gsem, isem):
    wid = lax.axis_index("s") * _NC + lax.axis_index("c")
    base = wid * _BPW
    lane = lax.iota(jnp.int32, _LANES)
    idx_copies = [
        pltpu.async_copy(
            idx_hbm.at[pl.ds(base + j * _CHUNK, _CHUNK)],
            idx_v.at[pl.ds(j * _CHUNK, _CHUNK)],
            isem,
        )
        for j in range(_NCHUNK)
    ]
    gathers = []
    for j in range(_NCHUNK):
        idx_copies[j].wait()
        for k in range(_CHUNK // _LANES):
            i = j * (_CHUNK // _LANES) + k
            row0 = base + i * _LANES
            r_term = ((row0 >> 7) << 10) + (row0 & 127)
            c = idx_v[pl.ds(i * _LANES, _LANES)]
            w = (
                lax.shift_left(lax.shift_right_logical(c, 3), 17)
                + lax.shift_left(lax.bitwise_and(c, 7), 7)
                + (lane + r_term)
            )
            flat_v[pl.ds(i * _LANES, _LANES)] = w
        gathers.append(
            pltpu.async_copy(
                table_hbm.at[flat_v.at[pl.ds(j * _CHUNK, _CHUNK)]],
                vals_v.at[pl.ds(j * _CHUNK, _CHUNK)],
                gsem,
            )
        )
    for cp in gathers:
        cp.wait()
    pltpu.sync_copy(vals_v, out_hbm.at[pl.ds(base, _BPW)])


def kernel(class_pred_softmax, class_max_prob_A_index):
    x = class_pred_softmax.T.reshape(_C // 8, 8, _B // 128, 128)
    x = x.transpose(0, 2, 1, 3)
    flat = x.reshape(_B * _C)
    idx = class_max_prob_A_index.astype(jnp.int32)
    return _match_class_sc(flat, idx)


# speedup vs baseline: 7.1288x; 1.0116x over previous
import functools

import jax
import jax.numpy as jnp
from jax import lax
from jax.experimental import pallas as pl
from jax.experimental.pallas import tpu as pltpu
from jax.experimental.pallas import tpu_sc as plsc

_B = 16384
_C = 1000
_NC = 2
_NS = 16
_NW = _NC * _NS
_BPW = _B // _NW
_CHUNK = 128
_NCHUNK = _BPW // _CHUNK
_LANES = 16


@functools.partial(
    pl.kernel,
    out_type=jax.ShapeDtypeStruct((_B,), jnp.float32),
    mesh=plsc.VectorSubcoreMesh(
        core_axis_name="c", subcore_axis_name="s",
        num_cores=_NC, num_subcores=_NS),
    scratch_types=[
        pltpu.VMEM((_BPW,), jnp.int32),
        pltpu.VMEM((_BPW,), jnp.int32),
        pltpu.VMEM((_BPW,), jnp.float32),
        pltpu.SemaphoreType.DMA,
        pltpu.SemaphoreType.DMA((_NCHUNK,)),
        pltpu.SemaphoreType.DMA((_NCHUNK,)),
    ],
)
def _match_class_sc(table_hbm, idx_hbm, out_hbm,
                    idx_v, flat_v, vals_v, isem, gsem, osem):
    wid = lax.axis_index("s") * _NC + lax.axis_index("c")
    base = wid * _BPW
    lane = lax.iota(jnp.int32, _LANES)
    pltpu.async_copy(idx_hbm.at[pl.ds(base, _BPW)], idx_v, isem).wait()
    gathers = []
    for j in range(_NCHUNK):
        def body(k, _, j=j):
            i = j * (_CHUNK // _LANES) + k
            row0 = base + i * _LANES
            r_term = ((row0 >> 7) << 10) + (row0 & 127)
            c = idx_v[pl.ds(i * _LANES, _LANES)]
            w = (
                lax.shift_left(lax.shift_right_logical(c, 3), 17)
                + lax.shift_left(lax.bitwise_and(c, 7), 7)
                + (lane + r_term)
            )
            flat_v[pl.ds(i * _LANES, _LANES)] = w
            return _
        lax.fori_loop(0, _CHUNK // _LANES, body, 0)
        gathers.append(
            pltpu.async_copy(
                table_hbm.at[flat_v.at[pl.ds(j * _CHUNK, _CHUNK)]],
                vals_v.at[pl.ds(j * _CHUNK, _CHUNK)],
                gsem.at[j],
            )
        )
    stores = []
    for j in range(_NCHUNK):
        gathers[j].wait()
        stores.append(
            pltpu.async_copy(
                vals_v.at[pl.ds(j * _CHUNK, _CHUNK)],
                out_hbm.at[pl.ds(base + j * _CHUNK, _CHUNK)],
                osem.at[j],
            )
        )
    for cp in stores:
        cp.wait()


def kernel(class_pred_softmax, class_max_prob_A_index):
    x = class_pred_softmax.T.reshape(_C // 8, 8, _B // 128, 128)
    x = x.transpose(0, 2, 1, 3)
    flat = x.reshape(_B * _C)
    idx = class_max_prob_A_index.astype(jnp.int32)
    return _match_class_sc(flat, idx)


# speedup vs baseline: 7.1321x; 1.0005x over previous
import functools

import jax
import jax.numpy as jnp
from jax import lax
from jax.experimental import pallas as pl
from jax.experimental.pallas import tpu as pltpu
from jax.experimental.pallas import tpu_sc as plsc

_B = 16384
_C = 1000
_NC = 2
_NS = 16
_NW = _NC * _NS
_BPW = _B // _NW
_CHUNK = 128
_NCHUNK = _BPW // _CHUNK
_LANES = 16


@functools.partial(
    pl.kernel,
    out_type=jax.ShapeDtypeStruct((_B,), jnp.float32),
    mesh=plsc.VectorSubcoreMesh(
        core_axis_name="c", subcore_axis_name="s",
        num_cores=_NC, num_subcores=_NS),
    scratch_types=[
        pltpu.VMEM((_BPW,), jnp.int32),
        pltpu.VMEM((_BPW,), jnp.int32),
        pltpu.VMEM((_BPW,), jnp.float32),
        pltpu.SemaphoreType.DMA,
        pltpu.SemaphoreType.DMA((_NCHUNK,)),
        pltpu.SemaphoreType.DMA((_NCHUNK,)),
    ],
    compiler_params=pltpu.CompilerParams(skip_device_barrier=True),
)
def _match_class_sc(table_hbm, idx_hbm, out_hbm,
                    idx_v, flat_v, vals_v, isem, gsem, osem):
    wid = lax.axis_index("s") * _NC + lax.axis_index("c")
    base = wid * _BPW
    lane = lax.iota(jnp.int32, _LANES)
    pltpu.async_copy(idx_hbm.at[pl.ds(base, _BPW)], idx_v, isem).wait()
    gathers = []
    for j in range(_NCHUNK):
        def body(k, _, j=j):
            i = j * (_CHUNK // _LANES) + k
            row0 = base + i * _LANES
            r_term = ((row0 >> 7) << 10) + (row0 & 127)
            c = idx_v[pl.ds(i * _LANES, _LANES)]
            w = (
                lax.shift_left(lax.shift_right_logical(c, 3), 17)
                + lax.shift_left(lax.bitwise_and(c, 7), 7)
                + (lane + r_term)
            )
            flat_v[pl.ds(i * _LANES, _LANES)] = w
            return _
        lax.fori_loop(0, _CHUNK // _LANES, body, 0)
        gathers.append(
            pltpu.async_copy(
                table_hbm.at[flat_v.at[pl.ds(j * _CHUNK, _CHUNK)]],
                vals_v.at[pl.ds(j * _CHUNK, _CHUNK)],
                gsem.at[j],
            )
        )
    stores = []
    for j in range(_NCHUNK):
        gathers[j].wait()
        stores.append(
            pltpu.async_copy(
                vals_v.at[pl.ds(j * _CHUNK, _CHUNK)],
                out_hbm.at[pl.ds(base + j * _CHUNK, _CHUNK)],
                osem.at[j],
            )
        )
    for cp in stores:
        cp.wait()


def kernel(class_pred_softmax, class_max_prob_A_index):
    x = class_pred_softmax.T.reshape(_C // 8, 8, _B // 128, 128)
    x = x.transpose(0, 2, 1, 3)
    flat = x.reshape(_B * _C)
    idx = class_max_prob_A_index.astype(jnp.int32)
    return _match_class_sc(flat, idx)
